# Initial kernel scaffold; baseline (speedup 1.0000x reference)
import jax
import jax.numpy as jnp
from jax.experimental import pallas as pl


def kernel(features, fea_bank, W_cls, b_cls, score_bank, trg_idx):
    raise NotImplementedError("write your pallas kernel here")


# speedup vs baseline: 31.1890x; 31.1890x over previous
import functools

import jax
import jax.numpy as jnp
from jax.experimental import pallas as pl
from jax.experimental.pallas import tpu as pltpu

_NEG = float("-inf")
_BIG = 2**30


def _top6(cv, ci):
    vs, js = [], []
    for _ in range(6):
        m = jnp.max(cv, axis=1, keepdims=True)
        isel = jnp.min(jnp.where(cv == m, ci, _BIG), axis=1, keepdims=True)
        vs.append(m)
        js.append(isel)
        cv = jnp.where((cv == m) & (ci == isel), _NEG, cv)
    return jnp.concatenate(vs, axis=1), jnp.concatenate(js, axis=1)


def _prep_body(feat_ref, w_ref, b_ref, tir_ref, tic_ref,
               so_ref, outf_ref, win_ref, disp_ref):
    f = feat_ref[...]
    B = f.shape[0]
    preds = jnp.dot(f, w_ref[...], preferred_element_type=jnp.float32)
    preds = preds + b_ref[...]
    m = jnp.max(preds, axis=1, keepdims=True)
    e = jnp.exp(preds - m)
    so = e / jnp.sum(e, axis=1, keepdims=True)
    so_ref[...] = so

    nrm = jnp.sqrt(jnp.sum(f * f, axis=1, keepdims=True))
    nrm = jnp.maximum(nrm, 1e-12)
    outf_ref[...] = f / nrm

    sv = jnp.sum(so, axis=0, keepdims=True)
    disp = (jnp.sum(sv * sv) - jnp.sum(so * so)) / float(B)
    disp_ref[...] = jnp.reshape(disp, (1, 1))

    tir = tir_ref[...]
    tic = tic_ref[...]
    row = jax.lax.broadcasted_iota(jnp.int32, (B, B), 0)
    col = jax.lax.broadcasted_iota(jnp.int32, (B, B), 1)
    eq = (tic == tir) & (row > col)
    loser = jnp.max(jnp.where(eq, 1, 0), axis=0, keepdims=True)
    win_ref[...] = 1 - loser


def _stream_body(outf_ref, bank_ref, mask_ref,
                 win_ref, tir_ref, vout_ref, iout_ref, rv_ref, ri_ref,
                 *, nb, bn):
    g = pl.program_id(0)

    @pl.when(g == 0)
    def _init():
        B = outf_ref.shape[0]
        rv_ref[...] = jnp.full((B, 8), _NEG, jnp.float32)
        ri_ref[...] = jnp.full((B, 8), -1, jnp.int32)

    q = outf_ref[...]
    B = q.shape[0]
    s = jax.lax.dot_general(q, bank_ref[...],
                            (((1,), (1,)), ((), ())),
                            preferred_element_type=jnp.float32)
    colg = g * bn + jax.lax.broadcasted_iota(jnp.int32, (B, bn), 1)
    s = jnp.where(mask_ref[...] > 0, _NEG, s)

    bvs, bis = [], []
    for _ in range(6):
        m = jnp.max(s, axis=1, keepdims=True)
        csel = jnp.min(jnp.where(s == m, colg, _BIG), axis=1, keepdims=True)
        bvs.append(m)
        bis.append(csel)
        s = jnp.where(colg == csel, _NEG, s)
    bv = jnp.concatenate(bvs, axis=1)
    bi = jnp.concatenate(bis, axis=1)

    pad2v = jnp.full((B, 2), _NEG, jnp.float32)
    pad2i = jnp.full((B, 2), -1, jnp.int32)
    cv = jnp.concatenate([rv_ref[...], bv, pad2v], axis=1)
    ci = jnp.concatenate([ri_ref[...], bi, pad2i], axis=1)
    nv, ni = _top6(cv, ci)
    rv_ref[...] = jnp.concatenate([nv, pad2v], axis=1)
    ri_ref[...] = jnp.concatenate([ni, pad2i], axis=1)

    @pl.when(g == nb - 1)
    def _finish():
        gc = jax.lax.dot_general(q, q,
                                 (((1,), (1,)), ((), ())),
                                 preferred_element_type=jnp.float32)
        winb = win_ref[...] > 0
        v = jnp.where(winb, gc, _NEG)
        colb = jax.lax.broadcasted_iota(jnp.int32, (B, B), 1)
        tir = tir_ref[...]
        pvs, pis = [], []
        for _ in range(6):
            m = jnp.max(v, axis=1, keepdims=True)
            bsel = jnp.min(jnp.where(v == m, colb, _BIG), axis=1, keepdims=True)
            jsel = jnp.max(jnp.where(colb == bsel, tir, -1), axis=1,
                           keepdims=True)
            pvs.append(m)
            pis.append(jsel)
            v = jnp.where(colb == bsel, _NEG, v)
        cv2 = jnp.concatenate([nv, jnp.concatenate(pvs, axis=1), pad2v, pad2v],
                              axis=1)
        ci2 = jnp.concatenate([ni, jnp.concatenate(pis, axis=1), pad2i, pad2i],
                              axis=1)
        fv, fi = _top6(cv2, ci2)
        vout_ref[...] = jnp.concatenate([fv, pad2v], axis=1)
        iout_ref[...] = jnp.concatenate([fi, pad2i], axis=1)


def _sel_body(if5_ref, tir_ref, win_ref, sel_ref):
    a = if5_ref[...]
    tir = tir_ref[...]
    B = tir.shape[1]
    ch = a.shape[0]
    col = jax.lax.broadcasted_iota(jnp.int32, (ch, B), 1)
    eq = (a == tir) & (win_ref[...] > 0)
    sel_ref[...] = jnp.max(jnp.where(eq, col, -1), axis=1, keepdims=True)


def _gather_body(if5_ref, sel_ref, sb0, sb1, sb2, sb3, sb4,
                 p0, p1, p2, p3, p4, soi_ref, out_ref, acc_ref, *, b, k):
    i = pl.program_id(0)

    @pl.when(i == 0)
    def _init():
        acc_ref[0, 0] = 0.0

    so_row = soi_ref[...]
    sbs = (sb0, sb1, sb2, sb3, sb4)
    ps = (p0, p1, p2, p3, p4)
    tot = jnp.zeros_like(so_row)
    for kk in range(k):
        patched = sel_ref[i * k + kk] >= 0
        row = jnp.where(patched, ps[kk][...], sbs[kk][...])
        tot = tot + row * (jnp.log(row) - so_row)
    acc_ref[0, 0] += jnp.sum(tot)

    @pl.when(i == b - 1)
    def _finish():
        out_ref[...] = jnp.reshape(acc_ref[0, 0] / float(b), (1, 1))


def kernel(features, fea_bank, W_cls, b_cls, score_bank, trg_idx):
    B, D = features.shape
    N = fea_bank.shape[0]
    C = W_cls.shape[1]
    K = 5
    ti = trg_idx.astype(jnp.int32)
    tir = ti.reshape(1, B)
    tic = ti.reshape(B, 1)

    so, outf, win, disp = pl.pallas_call(
        _prep_body,
        out_shape=[
            jax.ShapeDtypeStruct((B, C), jnp.float32),
            jax.ShapeDtypeStruct((B, D), jnp.float32),
            jax.ShapeDtypeStruct((1, B), jnp.int32),
            jax.ShapeDtypeStruct((1, 1), jnp.float32),
        ],
    )(features, W_cls, b_cls.reshape(1, C), tir, tic)

    BN = 512
    NB = -(-N // BN)
    npad = NB * BN

    maskp = jnp.zeros((1, npad), jnp.float32)
    maskp = maskp.at[0, N:].set(1.0)
    maskp = maskp.at[0, ti].set(1.0)

    vals6, idx6 = pl.pallas_call(
        functools.partial(_stream_body, nb=NB, bn=BN),
        grid=(NB,),
        in_specs=[
            pl.BlockSpec((B, D), lambda g: (0, 0)),
            pl.BlockSpec((BN, D), lambda g: (g, 0)),
            pl.BlockSpec((1, BN), lambda g: (0, g)),
            pl.BlockSpec((1, B), lambda g: (0, 0)),
            pl.BlockSpec((1, B), lambda g: (0, 0)),
        ],
        out_specs=[
            pl.BlockSpec((B, 8), lambda g: (0, 0)),
            pl.BlockSpec((B, 8), lambda g: (0, 0)),
        ],
        out_shape=[
            jax.ShapeDtypeStruct((B, 8), jnp.float32),
            jax.ShapeDtypeStruct((B, 8), jnp.int32),
        ],
        scratch_shapes=[
            pltpu.VMEM((B, 8), jnp.float32),
            pltpu.VMEM((B, 8), jnp.int32),
        ],
    )(outf, fea_bank, maskp, win, tir)

    if5 = idx6[:, 1:1 + K].reshape(B * K, 1)

    CH = 640
    sel = pl.pallas_call(
        _sel_body,
        grid=(B * K // CH,),
        in_specs=[
            pl.BlockSpec((CH, 1), lambda i: (i, 0)),
            pl.BlockSpec((1, B), lambda i: (0, 0)),
            pl.BlockSpec((1, B), lambda i: (0, 0)),
        ],
        out_specs=pl.BlockSpec((CH, 1), lambda i: (i, 0)),
        out_shape=jax.ShapeDtypeStruct((B * K, 1), jnp.int32),
    )(if5, tir, win)

    if5_flat = if5.reshape(B * K)
    sel_flat = sel.reshape(B * K)

    def _sb_map(kk):
        return lambda i, if5r, selr: (if5r[i * K + kk], 0, 0)

    def _p_map(kk):
        return lambda i, if5r, selr: (jnp.maximum(selr[i * K + kk], 0), 0, 0)

    sb3 = score_bank.reshape(N, 1, C)
    so3 = so.reshape(B, 1, C)
    att = pl.pallas_call(
        functools.partial(_gather_body, b=B, k=K),
        grid_spec=pltpu.PrefetchScalarGridSpec(
            num_scalar_prefetch=2,
            grid=(B,),
            in_specs=(
                [pl.BlockSpec((1, 1, C), _sb_map(kk)) for kk in range(K)]
                + [pl.BlockSpec((1, 1, C), _p_map(kk)) for kk in range(K)]
                + [pl.BlockSpec((1, 1, C), lambda i, if5r, selr: (i, 0, 0))]
            ),
            out_specs=pl.BlockSpec((1, 1), lambda i, if5r, selr: (0, 0)),
            scratch_shapes=[pltpu.SMEM((1, 1), jnp.float32)],
        ),
        out_shape=jax.ShapeDtypeStruct((1, 1), jnp.float32),
    )(if5_flat, sel_flat,
      sb3, sb3, sb3, sb3, sb3,
      so3, so3, so3, so3, so3, so3)

    return att[0, 0] + disp[0, 0]


# speedup vs baseline: 42.0629x; 1.3486x over previous
import functools

import jax
import jax.numpy as jnp
from jax.experimental import pallas as pl
from jax.experimental.pallas import tpu as pltpu

_NEG = float("-inf")
_BIG = 2**30


def _top6(cv, ci):
    vs, js = [], []
    for _ in range(6):
        m = jnp.max(cv, axis=1, keepdims=True)
        isel = jnp.min(jnp.where(cv == m, ci, _BIG), axis=1, keepdims=True)
        vs.append(m)
        js.append(isel)
        cv = jnp.where((cv == m) & (ci == isel), _NEG, cv)
    return jnp.concatenate(vs, axis=1), jnp.concatenate(js, axis=1)


def _prep_body(feat_ref, w_ref, b_ref, tir_ref, tic_ref,
               so_ref, outf_ref, win_ref, disp_ref):
    f = feat_ref[...]
    B = f.shape[0]
    preds = jnp.dot(f, w_ref[...], preferred_element_type=jnp.float32)
    preds = preds + b_ref[...]
    m = jnp.max(preds, axis=1, keepdims=True)
    e = jnp.exp(preds - m)
    so = e / jnp.sum(e, axis=1, keepdims=True)
    so_ref[...] = so

    nrm = jnp.sqrt(jnp.sum(f * f, axis=1, keepdims=True))
    nrm = jnp.maximum(nrm, 1e-12)
    outf_ref[...] = f / nrm

    sv = jnp.sum(so, axis=0, keepdims=True)
    disp = (jnp.sum(sv * sv) - jnp.sum(so * so)) / float(B)
    disp_ref[...] = jnp.reshape(disp, (1, 1))

    tir = tir_ref[...]
    tic = tic_ref[...]
    row = jax.lax.broadcasted_iota(jnp.int32, (B, B), 0)
    col = jax.lax.broadcasted_iota(jnp.int32, (B, B), 1)
    eq = (tic == tir) & (row > col)
    loser = jnp.max(jnp.where(eq, 1, 0), axis=0, keepdims=True)
    win_ref[...] = 1 - loser


def _stream_body(outf_ref, bank_ref, mask_ref,
                 win_ref, tir_ref, vout_ref, iout_ref, rv_ref, ri_ref,
                 *, nb, bn):
    g = pl.program_id(0)

    @pl.when(g == 0)
    def _init():
        B = outf_ref.shape[0]
        rv_ref[...] = jnp.full((B, 8), _NEG, jnp.float32)
        ri_ref[...] = jnp.full((B, 8), -1, jnp.int32)

    q = outf_ref[...]
    B = q.shape[0]
    s = jax.lax.dot_general(q, bank_ref[...],
                            (((1,), (1,)), ((), ())),
                            preferred_element_type=jnp.float32)
    colg = g * bn + jax.lax.broadcasted_iota(jnp.int32, (B, bn), 1)
    s = jnp.where(mask_ref[...] > 0, _NEG, s)

    bvs, bis = [], []
    for _ in range(6):
        m = jnp.max(s, axis=1, keepdims=True)
        csel = jnp.min(jnp.where(s == m, colg, _BIG), axis=1, keepdims=True)
        bvs.append(m)
        bis.append(csel)
        s = jnp.where(colg == csel, _NEG, s)
    bv = jnp.concatenate(bvs, axis=1)
    bi = jnp.concatenate(bis, axis=1)

    pad2v = jnp.full((B, 2), _NEG, jnp.float32)
    pad2i = jnp.full((B, 2), -1, jnp.int32)
    cv = jnp.concatenate([rv_ref[...], bv, pad2v], axis=1)
    ci = jnp.concatenate([ri_ref[...], bi, pad2i], axis=1)
    nv, ni = _top6(cv, ci)
    rv_ref[...] = jnp.concatenate([nv, pad2v], axis=1)
    ri_ref[...] = jnp.concatenate([ni, pad2i], axis=1)

    @pl.when(g == nb - 1)
    def _finish():
        gc = jax.lax.dot_general(q, q,
                                 (((1,), (1,)), ((), ())),
                                 preferred_element_type=jnp.float32)
        winb = win_ref[...] > 0
        v = jnp.where(winb, gc, _NEG)
        colb = jax.lax.broadcasted_iota(jnp.int32, (B, B), 1)
        tir = tir_ref[...]
        pvs, pis = [], []
        for _ in range(6):
            m = jnp.max(v, axis=1, keepdims=True)
            bsel = jnp.min(jnp.where(v == m, colb, _BIG), axis=1, keepdims=True)
            jsel = jnp.max(jnp.where(colb == bsel, tir, -1), axis=1,
                           keepdims=True)
            pvs.append(m)
            pis.append(jsel)
            v = jnp.where(colb == bsel, _NEG, v)
        cv2 = jnp.concatenate([nv, jnp.concatenate(pvs, axis=1), pad2v, pad2v],
                              axis=1)
        ci2 = jnp.concatenate([ni, jnp.concatenate(pis, axis=1), pad2i, pad2i],
                              axis=1)
        fv, fi = _top6(cv2, ci2)
        vout_ref[...] = jnp.concatenate([fv, pad2v], axis=1)
        iout_ref[...] = jnp.concatenate([fi, pad2i], axis=1)


def _sel_body(if5_ref, tir_ref, win_ref, sel_ref):
    a = if5_ref[...]
    tir = tir_ref[...]
    B = tir.shape[1]
    ch = a.shape[0]
    col = jax.lax.broadcasted_iota(jnp.int32, (ch, B), 1)
    eq = (a == tir) & (win_ref[...] > 0)
    sel_ref[...] = jnp.max(jnp.where(eq, col, -1), axis=1, keepdims=True)


def _gather_body(if5_ref, sel_ref, sb0, sb1, sb2, sb3, sb4,
                 p0, p1, p2, p3, p4, soi_ref, out_ref, acc_ref, *, b, k):
    i = pl.program_id(0)

    @pl.when(i == 0)
    def _init():
        acc_ref[0, 0] = 0.0

    so_row = soi_ref[...]
    sbs = (sb0, sb1, sb2, sb3, sb4)
    ps = (p0, p1, p2, p3, p4)
    tot = jnp.zeros_like(so_row)
    for kk in range(k):
        patched = sel_ref[i * k + kk] >= 0
        row = jnp.where(patched, ps[kk][...], sbs[kk][...])
        tot = tot + row * (jnp.log(row) - so_row)
    acc_ref[0, 0] += jnp.sum(tot)

    @pl.when(i == b - 1)
    def _finish():
        out_ref[...] = jnp.reshape(acc_ref[0, 0] / float(b), (1, 1))


def kernel(features, fea_bank, W_cls, b_cls, score_bank, trg_idx):
    B, D = features.shape
    N = fea_bank.shape[0]
    C = W_cls.shape[1]
    K = 5
    ti = trg_idx.astype(jnp.int32)
    tir = ti.reshape(1, B)
    tic = ti.reshape(B, 1)

    so, outf, win, disp = pl.pallas_call(
        _prep_body,
        out_shape=[
            jax.ShapeDtypeStruct((B, C), jnp.float32),
            jax.ShapeDtypeStruct((B, D), jnp.float32),
            jax.ShapeDtypeStruct((1, B), jnp.int32),
            jax.ShapeDtypeStruct((1, 1), jnp.float32),
        ],
    )(features, W_cls, b_cls.reshape(1, C), tir, tic)

    BN = 2048
    NB = -(-N // BN)
    npad = NB * BN

    maskp = jnp.zeros((1, npad), jnp.float32)
    maskp = maskp.at[0, N:].set(1.0)
    maskp = maskp.at[0, ti].set(1.0)

    vals6, idx6 = pl.pallas_call(
        functools.partial(_stream_body, nb=NB, bn=BN),
        grid=(NB,),
        in_specs=[
            pl.BlockSpec((B, D), lambda g: (0, 0)),
            pl.BlockSpec((BN, D), lambda g: (g, 0)),
            pl.BlockSpec((1, BN), lambda g: (0, g)),
            pl.BlockSpec((1, B), lambda g: (0, 0)),
            pl.BlockSpec((1, B), lambda g: (0, 0)),
        ],
        out_specs=[
            pl.BlockSpec((B, 8), lambda g: (0, 0)),
            pl.BlockSpec((B, 8), lambda g: (0, 0)),
        ],
        out_shape=[
            jax.ShapeDtypeStruct((B, 8), jnp.float32),
            jax.ShapeDtypeStruct((B, 8), jnp.int32),
        ],
        scratch_shapes=[
            pltpu.VMEM((B, 8), jnp.float32),
            pltpu.VMEM((B, 8), jnp.int32),
        ],
    )(outf, fea_bank, maskp, win, tir)

    if5 = idx6[:, 1:1 + K].reshape(B * K, 1)

    CH = 640
    sel = pl.pallas_call(
        _sel_body,
        grid=(B * K // CH,),
        in_specs=[
            pl.BlockSpec((CH, 1), lambda i: (i, 0)),
            pl.BlockSpec((1, B), lambda i: (0, 0)),
            pl.BlockSpec((1, B), lambda i: (0, 0)),
        ],
        out_specs=pl.BlockSpec((CH, 1), lambda i: (i, 0)),
        out_shape=jax.ShapeDtypeStruct((B * K, 1), jnp.int32),
    )(if5, tir, win)

    if5_flat = if5.reshape(B * K)
    sel_flat = sel.reshape(B * K)

    def _sb_map(kk):
        return lambda i, if5r, selr: (if5r[i * K + kk], 0, 0)

    def _p_map(kk):
        return lambda i, if5r, selr: (jnp.maximum(selr[i * K + kk], 0), 0, 0)

    sb3 = score_bank.reshape(N, 1, C)
    so3 = so.reshape(B, 1, C)
    att = pl.pallas_call(
        functools.partial(_gather_body, b=B, k=K),
        grid_spec=pltpu.PrefetchScalarGridSpec(
            num_scalar_prefetch=2,
            grid=(B,),
            in_specs=(
                [pl.BlockSpec((1, 1, C), _sb_map(kk)) for kk in range(K)]
                + [pl.BlockSpec((1, 1, C), _p_map(kk)) for kk in range(K)]
                + [pl.BlockSpec((1, 1, C), lambda i, if5r, selr: (i, 0, 0))]
            ),
            out_specs=pl.BlockSpec((1, 1), lambda i, if5r, selr: (0, 0)),
            scratch_shapes=[pltpu.SMEM((1, 1), jnp.float32)],
        ),
        out_shape=jax.ShapeDtypeStruct((1, 1), jnp.float32),
    )(if5_flat, sel_flat,
      sb3, sb3, sb3, sb3, sb3,
      so3, so3, so3, so3, so3, so3)

    return att[0, 0] + disp[0, 0]
